# baseline (speedup 1.0000x reference)
import functools

import functools

import jax
import jax.numpy as jnp
import numpy as np
from jax.experimental import pallas as pl
from jax.experimental.pallas import tpu as pltpu
from jax.experimental.pallas import tpu_sc as plsc

HIDDEN = 128
NUM_FILTERS = 128
NUM_INTERACTIONS = 6
NUM_GAUSSIANS = 50
CUTOFF = 10.0
MAX_NEIGHBORS = 16
N_NODES = 10000


def _ssp(x):
    return jax.nn.softplus(x) - jnp.log(2.0)


_KNN_R = 400
_BIG = 3.0e9
_BIGI = 1 << 30


_KNN_CH = 1280


def _knn_body(p_ref, pt_ref, nbr_ref, d2o_ref, scr):
    i = pl.program_id(0)
    p = p_ref[...]
    R = p.shape[0]
    Np = pt_ref.shape[1]
    CH = _KNN_CH
    NCH = Np // CH
    p16 = p.astype(jnp.bfloat16)
    sq_r = jnp.sum(p * p, axis=1, keepdims=True)
    for c in range(NCH):
        pt_c = pt_ref[:, c * CH:(c + 1) * CH]
        sq_c = jnp.sum(pt_c * pt_c, axis=0, keepdims=True)
        dot = jax.lax.dot_general(
            p16, pt_c.astype(jnp.bfloat16), (((1,), (0,)), ((), ())),
            preferred_element_type=jnp.float32)
        d2c = (sq_r + sq_c) - 2.0 * dot
        colg = jax.lax.broadcasted_iota(jnp.int32, (R, CH), 1) + c * CH
        rowg = jax.lax.broadcasted_iota(jnp.int32, (R, CH), 0) + i * R
        scr[:, c * CH:(c + 1) * CH] = jnp.where(colg == rowg, _BIG, d2c)
    nbrs, vals = [], []
    for _ in range(MAX_NEIGHBORS):
        cmins = [jnp.min(scr[:, c * CH:(c + 1) * CH], axis=1, keepdims=True)
                 for c in range(NCH)]
        m = jnp.min(jnp.concatenate(cmins, axis=1), axis=1, keepdims=True)
        am = None
        for c in range(NCH):
            d2c = scr[:, c * CH:(c + 1) * CH]
            colg = jax.lax.broadcasted_iota(jnp.int32, (R, CH), 1) + c * CH
            amc = jnp.min(jnp.where(d2c == m, colg, _BIGI), axis=1,
                          keepdims=True)
            am = amc if am is None else jnp.minimum(am, amc)
        for c in range(NCH):
            d2c = scr[:, c * CH:(c + 1) * CH]
            colg = jax.lax.broadcasted_iota(jnp.int32, (R, CH), 1) + c * CH
            scr[:, c * CH:(c + 1) * CH] = jnp.where(colg == am, _BIG, d2c)
        nbrs.append(am)
        vals.append(m)
    nbr_ref[...] = jnp.concatenate(nbrs, axis=1)
    d2o_ref[...] = jnp.concatenate(vals, axis=1)


def _knn(pos):
    N = pos.shape[0]
    R = _KNN_R
    Np = ((N + _KNN_CH - 1) // _KNN_CH) * _KNN_CH
    pos_pad = jnp.pad(pos, ((0, 0), (0, 5)))
    posT_pad = jnp.pad(pos_pad.T, ((0, 0), (0, Np - N)), constant_values=100.0)
    nbr, d2 = pl.pallas_call(
        _knn_body,
        grid=(N // R,),
        in_specs=[
            pl.BlockSpec((R, 8), lambda i: (i, 0)),
            pl.BlockSpec((8, Np), lambda i: (0, 0)),
        ],
        out_specs=[
            pl.BlockSpec((R, MAX_NEIGHBORS), lambda i: (i, 0)),
            pl.BlockSpec((R, MAX_NEIGHBORS), lambda i: (i, 0)),
        ],
        out_shape=[
            jax.ShapeDtypeStruct((N, MAX_NEIGHBORS), jnp.int32),
            jax.ShapeDtypeStruct((N, MAX_NEIGHBORS), jnp.float32),
        ],
        scratch_shapes=[pltpu.VMEM((R, Np), jnp.float32)],
    )(pos_pad, posT_pad)
    return nbr, d2


def _readout_kernel(h_ref, w1_ref, b1_ref, w2_ref, b2_ref, o_ref):
    t = _ssp(h_ref[...] @ w1_ref[...] + b1_ref[...])
    o_ref[...] = t @ w2_ref[...] + b2_ref[...]


def _dot16(a, b):
    return jax.lax.dot_general(
        a.astype(jnp.bfloat16), b.astype(jnp.bfloat16),
        (((1,), (0,)), ((), ())), preferred_element_type=jnp.float32)


def _sc_gather(hw, idx):
    E = idx.shape[0]
    WIN = 128
    mesh = plsc.VectorSubcoreMesh(core_axis_name="c", subcore_axis_name="s")
    idx2 = idx.reshape(1, E)

    @functools.partial(
        pl.kernel,
        out_type=jax.ShapeDtypeStruct((E, hw.shape[1]), hw.dtype),
        mesh=mesh)
    def gk(hw_hbm, i_hbm, o_hbm):
        def body(i_vmem, o_vmem):
            pltpu.sync_copy(hw_hbm.at[i_vmem.at[0]], o_vmem)

        pltpu.emit_pipeline(
            body,
            grid=(E // WIN,),
            in_specs=[pl.BlockSpec((1, WIN), index_map=lambda i: (0, i))],
            out_specs=[pl.BlockSpec((WIN, hw.shape[1]),
                                    index_map=lambda i: (i, 0))],
            core_axis_name=("c", "s"),
            dimension_semantics=(pltpu.PARALLEL,),
        )(i_hbm, o_hbm)

    return gk(hw, idx2)


_CFC_BN = 512


def _ew_body(pj_ref, p_ref, ew_ref):
    p = p_ref[...]
    for k in range(MAX_NEIGHBORS):
        dpos = pj_ref[k] - p
        d = jnp.sqrt(jnp.sum(dpos * dpos, axis=1, keepdims=True))
        ew_ref[k:k + 1, :] = jnp.transpose(d)


def _edge_weights(pj3, p128):
    N, H = p128.shape
    Bn = 1024
    return pl.pallas_call(
        _ew_body,
        grid=(N // Bn,),
        in_specs=[
            pl.BlockSpec((MAX_NEIGHBORS, Bn, H), lambda i: (0, i, 0)),
            pl.BlockSpec((Bn, H), lambda i: (i, 0)),
        ],
        out_specs=pl.BlockSpec((MAX_NEIGHBORS, Bn), lambda i: (0, i)),
        out_shape=jax.ShapeDtypeStruct((MAX_NEIGHBORS, N), jnp.float32),
    )(pj3, p128)


def _cfconv_body(xj_ref, ew_ref, off_ref, w1_ref, b1_ref, w2_ref,
                 b2_ref, agg_ref):
    off = off_ref[...]
    w1, b1 = w1_ref[...], b1_ref[...]
    w2, b2 = w2_ref[...], b2_ref[...]
    coeff = -0.5 / (CUTOFF / (NUM_GAUSSIANS - 1)) ** 2
    for k in range(MAX_NEIGHBORS):
        ew = jnp.transpose(ew_ref[k:k + 1, :])
        ea = jnp.exp(coeff * (ew - off) ** 2)
        t = _ssp(_dot16(ea, w1) + b1)
        Wk = _dot16(t, w2) + b2
        Ck = 0.5 * (jnp.cos(ew * (np.pi / CUTOFF)) + 1.0)
        msg = xj_ref[k] * (Wk * Ck)
        if k == 0:
            agg_ref[...] = msg
        else:
            agg_ref[...] += msg


def _cfconv(xj3, ew3, off, w1, b1, w2, b2):
    N = ew3.shape[1]
    Bn = _CFC_BN
    H = xj3.shape[2]
    return pl.pallas_call(
        _cfconv_body,
        grid=(N // Bn,),
        in_specs=[
            pl.BlockSpec((MAX_NEIGHBORS, Bn, H), lambda i: (0, i, 0)),
            pl.BlockSpec((MAX_NEIGHBORS, Bn), lambda i: (0, i)),
            pl.BlockSpec((1, NUM_GAUSSIANS), lambda i: (0, 0)),
            pl.BlockSpec((NUM_GAUSSIANS, NUM_FILTERS), lambda i: (0, 0)),
            pl.BlockSpec((1, NUM_FILTERS), lambda i: (0, 0)),
            pl.BlockSpec((NUM_FILTERS, H), lambda i: (0, 0)),
            pl.BlockSpec((1, H), lambda i: (0, 0)),
        ],
        out_specs=pl.BlockSpec((Bn, H), lambda i: (i, 0)),
        out_shape=jax.ShapeDtypeStruct((N, H), jnp.float32),
    )(xj3, ew3, off, w1, b1, w2, b2)


def _post_body(h_ref, agg_ref, cw2_ref, cb2_ref, lw_ref, lb_ref, cw1n_ref,
               hn_ref, hw_ref):
    hc = _ssp(_dot16(agg_ref[...], cw2_ref[...]) + cb2_ref[...])
    hc = _dot16(hc, lw_ref[...]) + lb_ref[...]
    hn = h_ref[...] + hc
    hn_ref[...] = hn
    hw_ref[...] = _dot16(hn, cw1n_ref[...])


def _post(h, agg, cw2, cb2, lw, lb, cw1n):
    N, H = h.shape
    Bn = 2048
    full = lambda i: (0, 0)
    blk = lambda i: (i, 0)
    return pl.pallas_call(
        _post_body,
        grid=(N // Bn,),
        in_specs=[
            pl.BlockSpec((Bn, H), blk),
            pl.BlockSpec((Bn, H), blk),
            pl.BlockSpec((H, H), full),
            pl.BlockSpec((1, H), full),
            pl.BlockSpec((H, H), full),
            pl.BlockSpec((1, H), full),
            pl.BlockSpec((H, H), full),
        ],
        out_specs=[pl.BlockSpec((Bn, H), blk), pl.BlockSpec((Bn, H), blk)],
        out_shape=[jax.ShapeDtypeStruct((N, H), jnp.float32),
                   jax.ShapeDtypeStruct((N, H), jnp.float32)],
    )(h, agg, cw2, cb2, lw, lb, cw1n)


def _embed_body(z_ref, emb_ref, cw1_ref, h_ref, hw_ref):
    z = z_ref[...]
    ids = jax.lax.broadcasted_iota(jnp.int32, (1, 100), 1)
    oh = (z == ids).astype(jnp.float32)
    h = jax.lax.dot_general(oh, emb_ref[...], (((1,), (0,)), ((), ())),
                            precision=jax.lax.Precision.HIGHEST,
                            preferred_element_type=jnp.float32)
    h_ref[...] = h
    hw_ref[...] = _dot16(h, cw1_ref[...])


def _embed(z, emb, cw1_0):
    N = z.shape[0]
    H = emb.shape[1]
    Bn = 2048
    return pl.pallas_call(
        _embed_body,
        grid=(N // Bn,),
        in_specs=[
            pl.BlockSpec((Bn, 1), lambda i: (i, 0)),
            pl.BlockSpec((100, H), lambda i: (0, 0)),
            pl.BlockSpec((H, H), lambda i: (0, 0)),
        ],
        out_specs=[pl.BlockSpec((Bn, H), lambda i: (i, 0)),
                   pl.BlockSpec((Bn, H), lambda i: (i, 0))],
        out_shape=[jax.ShapeDtypeStruct((N, H), jnp.float32),
                   jax.ShapeDtypeStruct((N, H), jnp.float32)],
    )(z.reshape(N, 1).astype(jnp.int32), emb, cw1_0)


def kernel(z, pos, emb, mlp_w1, mlp_b1, mlp_w2, mlp_b2, conv_w1, conv_w2,
           conv_b2, lin_w, lin_b, out_w1, out_b1, out_w2, out_b2):
    N = pos.shape[0]
    nbr, d2 = _knn(pos)
    Np = 10240
    idx = jnp.pad(nbr.T, ((0, 0), (0, Np - N))).reshape(-1)
    offset = jnp.linspace(0.0, CUTOFF, NUM_GAUSSIANS).reshape(1, -1)
    p128 = jnp.pad(pos, ((0, Np - N), (0, 125)))
    pj3 = _sc_gather(p128, idx).reshape(MAX_NEIGHBORS, Np, 128)
    ew3 = _edge_weights(pj3, p128)

    zp = jnp.pad(z, (0, Np - N))
    h, hw = _embed(zp, emb, conv_w1[0])
    for l in range(NUM_INTERACTIONS):
        xj = _sc_gather(hw, idx).reshape(MAX_NEIGHBORS, Np, HIDDEN)
        agg = _cfconv(xj, ew3, offset, mlp_w1[l],
                      mlp_b1[l].reshape(1, -1),
                      mlp_w2[l], mlp_b2[l].reshape(1, -1))
        nxt = conv_w1[l + 1] if l + 1 < NUM_INTERACTIONS else conv_w1[0]
        h, hw = _post(h, agg, conv_w2[l], conv_b2[l].reshape(1, -1),
                      lin_w[l], lin_b[l].reshape(1, -1), nxt)

    out = pl.pallas_call(
        _readout_kernel,
        out_shape=jax.ShapeDtypeStruct((Np, 1), jnp.float32),
    )(h, out_w1, out_b1[None, :], out_w2, out_b2[None, :])
    return out[:N]


# speedup vs baseline: 1.0105x; 1.0105x over previous
import functools

import functools

import jax
import jax.numpy as jnp
import numpy as np
from jax.experimental import pallas as pl
from jax.experimental.pallas import tpu as pltpu
from jax.experimental.pallas import tpu_sc as plsc

HIDDEN = 128
NUM_FILTERS = 128
NUM_INTERACTIONS = 6
NUM_GAUSSIANS = 50
CUTOFF = 10.0
MAX_NEIGHBORS = 16
N_NODES = 10000


def _ssp(x):
    return jax.nn.softplus(x) - jnp.log(2.0)


_KNN_R = 400
_BIG = 3.0e9
_BIGI = 1 << 30


_KNN_CH = 1280


def _knn_body(p_ref, pt_ref, nbr_ref, d2o_ref, scr):
    i = pl.program_id(0)
    p = p_ref[...]
    R = p.shape[0]
    Np = pt_ref.shape[1]
    CH = _KNN_CH
    NCH = Np // CH
    p16 = p.astype(jnp.bfloat16)
    sq_r = jnp.sum(p * p, axis=1, keepdims=True)
    for c in range(NCH):
        pt_c = pt_ref[:, c * CH:(c + 1) * CH]
        sq_c = jnp.sum(pt_c * pt_c, axis=0, keepdims=True)
        dot = jax.lax.dot_general(
            p16, pt_c.astype(jnp.bfloat16), (((1,), (0,)), ((), ())),
            preferred_element_type=jnp.float32)
        d2c = (sq_r + sq_c) - 2.0 * dot
        colg = jax.lax.broadcasted_iota(jnp.int32, (R, CH), 1) + c * CH
        rowg = jax.lax.broadcasted_iota(jnp.int32, (R, CH), 0) + i * R
        scr[:, c * CH:(c + 1) * CH] = jnp.where(colg == rowg, _BIG, d2c)
    nbrs, vals = [], []
    for _ in range(MAX_NEIGHBORS):
        cmins = [jnp.min(scr[:, c * CH:(c + 1) * CH], axis=1, keepdims=True)
                 for c in range(NCH)]
        m = jnp.min(jnp.concatenate(cmins, axis=1), axis=1, keepdims=True)
        am = None
        for c in range(NCH):
            d2c = scr[:, c * CH:(c + 1) * CH]
            colg = jax.lax.broadcasted_iota(jnp.int32, (R, CH), 1) + c * CH
            amc = jnp.min(jnp.where(d2c == m, colg, _BIGI), axis=1,
                          keepdims=True)
            am = amc if am is None else jnp.minimum(am, amc)
        for c in range(NCH):
            d2c = scr[:, c * CH:(c + 1) * CH]
            colg = jax.lax.broadcasted_iota(jnp.int32, (R, CH), 1) + c * CH
            scr[:, c * CH:(c + 1) * CH] = jnp.where(colg == am, _BIG, d2c)
        nbrs.append(am)
        vals.append(m)
    nbr_ref[...] = jnp.concatenate(nbrs, axis=1)
    d2o_ref[...] = jnp.concatenate(vals, axis=1)


def _knn(pos):
    N = pos.shape[0]
    R = _KNN_R
    Np = ((N + _KNN_CH - 1) // _KNN_CH) * _KNN_CH
    pos_pad = jnp.pad(pos, ((0, 0), (0, 5)))
    posT_pad = jnp.pad(pos_pad.T, ((0, 0), (0, Np - N)), constant_values=100.0)
    nbr, d2 = pl.pallas_call(
        _knn_body,
        grid=(N // R,),
        in_specs=[
            pl.BlockSpec((R, 8), lambda i: (i, 0)),
            pl.BlockSpec((8, Np), lambda i: (0, 0)),
        ],
        out_specs=[
            pl.BlockSpec((R, MAX_NEIGHBORS), lambda i: (i, 0)),
            pl.BlockSpec((R, MAX_NEIGHBORS), lambda i: (i, 0)),
        ],
        out_shape=[
            jax.ShapeDtypeStruct((N, MAX_NEIGHBORS), jnp.int32),
            jax.ShapeDtypeStruct((N, MAX_NEIGHBORS), jnp.float32),
        ],
        scratch_shapes=[pltpu.VMEM((R, Np), jnp.float32)],
    )(pos_pad, posT_pad)
    return nbr, d2


def _readout_kernel(h_ref, w1_ref, b1_ref, w2_ref, b2_ref, o_ref):
    t = _ssp(h_ref[...] @ w1_ref[...] + b1_ref[...])
    o_ref[...] = t @ w2_ref[...] + b2_ref[...]


def _dot16(a, b):
    return jax.lax.dot_general(
        a.astype(jnp.bfloat16), b.astype(jnp.bfloat16),
        (((1,), (0,)), ((), ())), preferred_element_type=jnp.float32)


def _sc_gather(hw, idx, win=256):
    E = idx.shape[0]
    WIN = win
    mesh = plsc.VectorSubcoreMesh(core_axis_name="c", subcore_axis_name="s")
    idx2 = idx.reshape(1, E)

    @functools.partial(
        pl.kernel,
        out_type=jax.ShapeDtypeStruct((E, hw.shape[1]), hw.dtype),
        mesh=mesh)
    def gk(hw_hbm, i_hbm, o_hbm):
        def body(i_vmem, o_vmem):
            pltpu.sync_copy(hw_hbm.at[i_vmem.at[0]], o_vmem)

        pltpu.emit_pipeline(
            body,
            grid=(E // WIN,),
            in_specs=[pl.BlockSpec((1, WIN), index_map=lambda i: (0, i))],
            out_specs=[pl.BlockSpec((WIN, hw.shape[1]),
                                    index_map=lambda i: (i, 0))],
            core_axis_name=("c", "s"),
            dimension_semantics=(pltpu.PARALLEL,),
        )(i_hbm, o_hbm)

    return gk(hw, idx2)


_CFC_BN = 512


def _ew_body(pj_ref, p_ref, ew_ref):
    p = p_ref[...]
    for k in range(MAX_NEIGHBORS):
        dpos = pj_ref[k] - p
        d = jnp.sqrt(jnp.sum(dpos * dpos, axis=1, keepdims=True))
        ew_ref[k:k + 1, :] = jnp.transpose(d)


def _edge_weights(pj3, p128):
    N, H = p128.shape
    Bn = 1024
    return pl.pallas_call(
        _ew_body,
        grid=(N // Bn,),
        in_specs=[
            pl.BlockSpec((MAX_NEIGHBORS, Bn, H), lambda i: (0, i, 0)),
            pl.BlockSpec((Bn, H), lambda i: (i, 0)),
        ],
        out_specs=pl.BlockSpec((MAX_NEIGHBORS, Bn), lambda i: (0, i)),
        out_shape=jax.ShapeDtypeStruct((MAX_NEIGHBORS, N), jnp.float32),
    )(pj3, p128)


def _cfconv_body(xj_ref, ew_ref, off_ref, w1_ref, b1_ref, w2_ref,
                 b2_ref, agg_ref):
    off = off_ref[...]
    w1, b1 = w1_ref[...], b1_ref[...]
    w2, b2 = w2_ref[...], b2_ref[...]
    coeff = -0.5 / (CUTOFF / (NUM_GAUSSIANS - 1)) ** 2
    for k in range(MAX_NEIGHBORS):
        ew = jnp.transpose(ew_ref[k:k + 1, :])
        ea = jnp.exp(coeff * (ew - off) ** 2)
        t = _ssp(_dot16(ea, w1) + b1)
        Wk = _dot16(t, w2) + b2
        Ck = 0.5 * (jnp.cos(ew * (np.pi / CUTOFF)) + 1.0)
        msg = xj_ref[k].astype(jnp.float32) * (Wk * Ck)
        if k == 0:
            agg_ref[...] = msg
        else:
            agg_ref[...] += msg


def _cfconv(xj3, ew3, off, w1, b1, w2, b2):
    N = ew3.shape[1]
    Bn = _CFC_BN
    H = xj3.shape[2]
    return pl.pallas_call(
        _cfconv_body,
        grid=(N // Bn,),
        in_specs=[
            pl.BlockSpec((MAX_NEIGHBORS, Bn, H), lambda i: (0, i, 0)),
            pl.BlockSpec((MAX_NEIGHBORS, Bn), lambda i: (0, i)),
            pl.BlockSpec((1, NUM_GAUSSIANS), lambda i: (0, 0)),
            pl.BlockSpec((NUM_GAUSSIANS, NUM_FILTERS), lambda i: (0, 0)),
            pl.BlockSpec((1, NUM_FILTERS), lambda i: (0, 0)),
            pl.BlockSpec((NUM_FILTERS, H), lambda i: (0, 0)),
            pl.BlockSpec((1, H), lambda i: (0, 0)),
        ],
        out_specs=pl.BlockSpec((Bn, H), lambda i: (i, 0)),
        out_shape=jax.ShapeDtypeStruct((N, H), jnp.float32),
    )(xj3, ew3, off, w1, b1, w2, b2)


def _post_body(h_ref, agg_ref, cw2_ref, cb2_ref, lw_ref, lb_ref, cw1n_ref,
               hn_ref, hw_ref):
    hc = _ssp(_dot16(agg_ref[...], cw2_ref[...]) + cb2_ref[...])
    hc = _dot16(hc, lw_ref[...]) + lb_ref[...]
    hn = h_ref[...] + hc
    hn_ref[...] = hn
    hw_ref[...] = _dot16(hn, cw1n_ref[...])


def _post(h, agg, cw2, cb2, lw, lb, cw1n):
    N, H = h.shape
    Bn = 2048
    full = lambda i: (0, 0)
    blk = lambda i: (i, 0)
    return pl.pallas_call(
        _post_body,
        grid=(N // Bn,),
        in_specs=[
            pl.BlockSpec((Bn, H), blk),
            pl.BlockSpec((Bn, H), blk),
            pl.BlockSpec((H, H), full),
            pl.BlockSpec((1, H), full),
            pl.BlockSpec((H, H), full),
            pl.BlockSpec((1, H), full),
            pl.BlockSpec((H, H), full),
        ],
        out_specs=[pl.BlockSpec((Bn, H), blk), pl.BlockSpec((Bn, H), blk)],
        out_shape=[jax.ShapeDtypeStruct((N, H), jnp.float32),
                   jax.ShapeDtypeStruct((N, H), jnp.float32)],
    )(h, agg, cw2, cb2, lw, lb, cw1n)


def _embed_body(z_ref, emb_ref, cw1_ref, h_ref, hw_ref):
    z = z_ref[...]
    ids = jax.lax.broadcasted_iota(jnp.int32, (1, 100), 1)
    oh = (z == ids).astype(jnp.float32)
    h = jax.lax.dot_general(oh, emb_ref[...], (((1,), (0,)), ((), ())),
                            precision=jax.lax.Precision.HIGHEST,
                            preferred_element_type=jnp.float32)
    h_ref[...] = h
    hw_ref[...] = _dot16(h, cw1_ref[...])


def _embed(z, emb, cw1_0):
    N = z.shape[0]
    H = emb.shape[1]
    Bn = 2048
    return pl.pallas_call(
        _embed_body,
        grid=(N // Bn,),
        in_specs=[
            pl.BlockSpec((Bn, 1), lambda i: (i, 0)),
            pl.BlockSpec((100, H), lambda i: (0, 0)),
            pl.BlockSpec((H, H), lambda i: (0, 0)),
        ],
        out_specs=[pl.BlockSpec((Bn, H), lambda i: (i, 0)),
                   pl.BlockSpec((Bn, H), lambda i: (i, 0))],
        out_shape=[jax.ShapeDtypeStruct((N, H), jnp.float32),
                   jax.ShapeDtypeStruct((N, H), jnp.float32)],
    )(z.reshape(N, 1).astype(jnp.int32), emb, cw1_0)


def kernel(z, pos, emb, mlp_w1, mlp_b1, mlp_w2, mlp_b2, conv_w1, conv_w2,
           conv_b2, lin_w, lin_b, out_w1, out_b1, out_w2, out_b2):
    N = pos.shape[0]
    nbr, d2 = _knn(pos)
    Np = 10240
    idx = jnp.pad(nbr.T, ((0, 0), (0, Np - N))).reshape(-1)
    offset = jnp.linspace(0.0, CUTOFF, NUM_GAUSSIANS).reshape(1, -1)
    p128 = jnp.pad(pos, ((0, Np - N), (0, 125)))
    pj3 = _sc_gather(p128, idx, win=256).reshape(MAX_NEIGHBORS, Np, 128)
    ew3 = _edge_weights(pj3, p128)

    zp = jnp.pad(z, (0, Np - N))
    h, hw = _embed(zp, emb, conv_w1[0])
    for l in range(NUM_INTERACTIONS):
        xj = _sc_gather(hw, idx).reshape(MAX_NEIGHBORS, Np, HIDDEN)
        agg = _cfconv(xj, ew3, offset, mlp_w1[l],
                      mlp_b1[l].reshape(1, -1),
                      mlp_w2[l], mlp_b2[l].reshape(1, -1))
        nxt = conv_w1[l + 1] if l + 1 < NUM_INTERACTIONS else conv_w1[0]
        h, hw = _post(h, agg, conv_w2[l], conv_b2[l].reshape(1, -1),
                      lin_w[l], lin_b[l].reshape(1, -1), nxt)

    out = pl.pallas_call(
        _readout_kernel,
        out_shape=jax.ShapeDtypeStruct((Np, 1), jnp.float32),
    )(h, out_w1, out_b1[None, :], out_w2, out_b2[None, :])
    return out[:N]
